# Initial kernel scaffold; baseline (speedup 1.0000x reference)
import jax
import jax.numpy as jnp
from jax.experimental import pallas as pl


def kernel(x, edge_index, edge_type, W_rel, loop_weight, reverse):
    raise NotImplementedError("write your pallas kernel here")


# speedup vs baseline: 1.6767x; 1.6767x over previous
import functools

import jax
import jax.numpy as jnp
from jax import lax
from jax.experimental import pallas as pl
from jax.experimental.pallas import tpu as pltpu
from jax.experimental.pallas import tpu_sc as plsc

N = 10000
E = 320000
D = 128
R = 8

DP = 144
BN = 256
NP = 10240
GI = NP // BN
NH = NP // 2
NCORE = 2
NSUB = 16
NPASS = R
EPW = E // NSUB
SB = 800
NBLK = EPW // SB
NGRP = SB // 16
BT = 64
CAP = 960
PK = 16384
TRASH = NH
ZR = NH // NSUB

_mesh = plsc.VectorSubcoreMesh(
    core_axis_name="c", subcore_axis_name="s", num_cores=NCORE, num_subcores=NSUB
)


@functools.partial(
    pl.kernel,
    out_type=[
        jax.ShapeDtypeStruct((R * NP, DP), jnp.float32),
    ],
    mesh=_mesh,
    compiler_params=pltpu.CompilerParams(use_tc_tiling_on_sc=False),
    scratch_types=[
        pltpu.VMEM((SB,), jnp.int32),
        pltpu.VMEM((SB,), jnp.int32),
        pltpu.VMEM((SB,), jnp.int32),
        pltpu.VMEM((CAP,), jnp.int32),
        pltpu.VMEM((8, 16), jnp.int32),
        pltpu.VMEM((1, BT), jnp.int32),
        pltpu.VMEM((1, BT), jnp.int32),
        pltpu.VMEM((BT, DP), jnp.float32),
        pltpu.VMEM_SHARED((NH + 8, DP), jnp.float32),
        pltpu.SemaphoreType.DMA,
    ],
)
def _sc_segsum(src_hbm, dst_hbm, typ_hbm, x_hbm, tab_hbm, agg_out,
               st_src, st_dst, st_typ, pk_buf, tab_v,
               idx_src, idx_loc, rows, acc, sem):
    c = lax.axis_index("c")
    s = lax.axis_index("s")

    lanes = lax.iota(jnp.int32, 16)
    onevec = jnp.where(lanes == 0, 1.0, 0.0).astype(jnp.float32)
    zvec = jnp.zeros((16,), jnp.float32)

    _dn = lax.GatherDimensionNumbers(
        offset_dims=(), collapsed_slice_dims=(0,), start_index_map=(0,))

    def _gather16(v, idx):
        return lax.gather(v, idx[:, None], _dn, slice_sizes=(1,),
                          mode=lax.GatherScatterMode.PROMISE_IN_BOUNDS)

    def _prefix16(v):
        for k in (1, 2, 4, 8):
            sh = _gather16(v, jnp.maximum(lanes - k, 0))
            v = v + jnp.where(lanes >= k, sh, 0)
        return v

    def _compress16(pref, vals):
        tgt = lanes + 1
        pos = jnp.zeros((16,), jnp.int32)
        for k in (8, 4, 2, 1):
            idx = jnp.minimum(pos + (k - 1), 15)
            cond = _gather16(pref, idx) < tgt
            pos = pos + jnp.where(cond, k, 0)
        return _gather16(vals, pos)

    base = s * ZR
    trashv = jnp.full((16,), TRASH, jnp.int32)

    def _fire(j, carry):
        jb = j * BT
        for jj in range(BT // 16):
            pk = pk_buf[pl.ds(jb + 16 * jj, 16)]
            idx_src[0, pl.ds(16 * jj, 16)] = lax.shift_right_logical(pk, 14)
            idx_loc[0, pl.ds(16 * jj, 16)] = pk & (PK - 1)
        pltpu.async_copy(x_hbm.at[idx_src.at[0]], rows, sem).wait()
        pltpu.sync_copy(rows, acc.at[idx_loc.at[0]], add=True)
        return carry

    def _zero_rows(i, carry):
        for jj in range(DP // 16):
            rows[i, pl.ds(16 * jj, 16)] = zvec
        return carry

    def _pass(t, carry):
        pltpu.sync_copy(tab_hbm.at[pl.ds(c * 64 + t * 8, 8)], tab_v)
        lax.fori_loop(0, BT, _zero_rows, 0)
        for off in range(0, ZR, BT):
            pltpu.sync_copy(rows, acc.at[pl.ds(base + off, BT)])
        plsc.subcore_barrier()

        def _block(blk, cursor):
            ebase = s * EPW + blk * SB
            pltpu.sync_copy(src_hbm.at[pl.ds(ebase, SB)], st_src)
            pltpu.sync_copy(dst_hbm.at[pl.ds(ebase, SB)], st_dst)
            pltpu.sync_copy(typ_hbm.at[pl.ds(ebase, SB)], st_typ)

            def _grp(g, cur):
                off = g * 16
                tv = st_typ[pl.ds(off, 16)]
                dvl = st_dst[pl.ds(off, 16)] - tab_v[1, :]
                pk = st_src[pl.ds(off, 16)] * PK + dvl
                m = (tv == tab_v[0, :]) & (dvl >= 0) & (dvl < NH)
                pr = _prefix16(jnp.where(m, 1, 0))
                pk_buf[pl.ds(cur, 16)] = _compress16(pr, pk)
                return cur + pr[15]

            cursor = lax.fori_loop(0, NGRP, _grp, cursor)

            nb = cursor // BT
            lax.fori_loop(0, nb, _fire, 0)
            rs = nb * BT
            for i in range(BT // 16):
                tmp = pk_buf[pl.ds(rs + 16 * i, 16)]
                pk_buf[pl.ds(16 * i, 16)] = tmp
            return cursor - rs

        cursor = lax.fori_loop(0, NBLK, _block, jnp.int32(0))

        for i in range(BT // 16):
            pk_buf[pl.ds(cursor + 16 * i, 16)] = trashv
        nlast = (cursor + BT - 1) // BT
        lax.fori_loop(0, nlast, _fire, 0)

        plsc.subcore_barrier()
        rel = (t // 2) * NCORE + c
        half = t - (t // 2) * 2
        hb = rel * NP + half * NH + base
        for off in range(0, ZR, BT):
            pltpu.sync_copy(acc.at[pl.ds(base + off, BT)], rows)
            pltpu.sync_copy(rows, agg_out.at[pl.ds(hb + off, BT)])
        plsc.subcore_barrier()
        return carry

    lax.fori_loop(0, NPASS, _pass, 0)


def _tc_body(agg_ref, x_ref, wr_ref, lw_ref, out_ref):
    r = pl.program_id(1)
    a = agg_ref[...]
    scale = 1.0 / jnp.maximum(a[:, 128:129], 1.0)
    h = jnp.dot(a[:, :128] * scale, wr_ref[0],
                preferred_element_type=jnp.float32)

    @pl.when(r == 0)
    def _():
        out_ref[...] = h + jnp.dot(x_ref[...][:, :128], lw_ref[...],
                                   preferred_element_type=jnp.float32)

    @pl.when(r > 0)
    def _():
        out_ref[...] = out_ref[...] + h


def _tc_combine(agg, x_pad, W_rel, loop_weight):
    return pl.pallas_call(
        _tc_body,
        grid=(GI, R),
        in_specs=[
            pl.BlockSpec((BN, DP), lambda i, r: (r * GI + i, 0)),
            pl.BlockSpec((BN, DP), lambda i, r: (i, 0)),
            pl.BlockSpec((1, D, D), lambda i, r: (r, 0, 0)),
            pl.BlockSpec((D, D), lambda i, r: (0, 0)),
        ],
        out_specs=pl.BlockSpec((BN, D), lambda i, r: (i, 0)),
        out_shape=jax.ShapeDtypeStruct((NP, D), jnp.float32),
    )(agg, x_pad, W_rel, loop_weight)


def _pass_table():
    import numpy as np
    tab = np.zeros((NCORE * NPASS * 8, 16), np.int32)
    for c in range(NCORE):
        for t in range(NPASS):
            rel = (t // 2) * NCORE + c
            half = t % 2
            tab[c * 64 + t * 8 + 0, :] = rel
            tab[c * 64 + t * 8 + 1, :] = half * NH
    return jnp.asarray(tab)


def kernel(x, edge_index, edge_type, W_rel, loop_weight, reverse):
    x = x.astype(jnp.float32)
    ei = edge_index.astype(jnp.int32)
    et = edge_type.astype(jnp.int32)
    rev = jnp.asarray(reverse, jnp.int32)
    src = jnp.where(rev != 0, ei[1], ei[0])
    dst = jnp.where(rev != 0, ei[0], ei[1])
    x_pad = jnp.zeros((NP, DP), jnp.float32).at[:N, :D].set(x)
    x_pad = x_pad.at[:, D].set(1.0)
    agg = _sc_segsum(src, dst, et, x_pad, _pass_table())[0]
    out = _tc_combine(agg, x_pad,
                      W_rel.astype(jnp.float32), loop_weight.astype(jnp.float32))
    return out[:N]
